# baseline (speedup 1.0000x reference)
import functools

import jax
import jax.numpy as jnp
from jax import lax
from jax.experimental import pallas as pl
from jax.experimental.pallas import tpu as pltpu
from jax.experimental.pallas import tpu_sc as plsc

NW = 32
CHUNK = 128
NBUF = 4


def kernel(x, table):
    B, H = x.shape
    V, D = table.shape
    total = B * H
    per_w = total // NW
    n_ch = per_w // CHUNK

    x_resh = x.reshape(NW, n_ch, CHUNK).astype(jnp.int32)
    mesh = plsc.VectorSubcoreMesh(core_axis_name="c", subcore_axis_name="s")

    @functools.partial(
        pl.kernel,
        out_type=jax.ShapeDtypeStruct((total, D), jnp.float32),
        mesh=mesh,
        scratch_types=[
            pltpu.VMEM((n_ch, CHUNK), jnp.int32),
            pltpu.VMEM((NBUF, CHUNK, D), jnp.float32),
            pltpu.SemaphoreType.DMA,
            pltpu.SemaphoreType.DMA,
        ],
        compiler_params=pltpu.CompilerParams(use_tc_tiling_on_sc=False),
    )
    def emb(x_hbm, table_hbm, out_hbm, idx_v, buf, sem_g, sem_w):
        wid = lax.axis_index("s") * 2 + lax.axis_index("c")
        base = wid * per_w
        pltpu.sync_copy(x_hbm.at[wid], idx_v)

        def group(g, carry):
            gathers = []
            for b in range(NBUF):
                j = g * NBUF + b
                gathers.append(
                    pltpu.async_copy(table_hbm.at[idx_v.at[j]], buf.at[b], sem_g)
                )
            writes = []
            for b in range(NBUF):
                j = g * NBUF + b
                gathers[b].wait()
                writes.append(
                    pltpu.async_copy(
                        buf.at[b], out_hbm.at[pl.ds(base + j * CHUNK, CHUNK)], sem_w
                    )
                )
            for b in range(NBUF):
                writes[b].wait()
            return carry

        lax.fori_loop(0, n_ch // NBUF, group, 0)

    out = emb(x_resh, table)
    return out.reshape(B, H, D)


# speedup vs baseline: 1.0033x; 1.0033x over previous
import functools

import jax
import jax.numpy as jnp
from jax import lax
from jax.experimental import pallas as pl
from jax.experimental.pallas import tpu as pltpu
from jax.experimental.pallas import tpu_sc as plsc

NW = 32
ROWS = 256
NBUF = 5


def kernel(x, table):
    B, H = x.shape
    V, D = table.shape
    total = B * H
    per_w = total // NW
    n_ch = per_w // ROWS

    x_resh = x.reshape(NW, per_w).astype(jnp.int32)
    mesh = plsc.VectorSubcoreMesh(core_axis_name="c", subcore_axis_name="s")

    @functools.partial(
        pl.kernel,
        out_type=jax.ShapeDtypeStruct((total, D), jnp.float32),
        mesh=mesh,
        scratch_types=[
            pltpu.VMEM((per_w,), jnp.int32),
            pltpu.VMEM((NBUF, ROWS, D), jnp.float32),
            pltpu.SemaphoreType.DMA,
            pltpu.SemaphoreType.DMA,
        ],
        compiler_params=pltpu.CompilerParams(use_tc_tiling_on_sc=False),
    )
    def emb(x_hbm, table_hbm, out_hbm, idx_v, buf, sem_g, sem_w):
        wid = lax.axis_index("s") * 2 + lax.axis_index("c")
        base = wid * per_w
        pltpu.sync_copy(x_hbm.at[wid], idx_v)

        def group(g, carry):
            gathers = []
            for b in range(NBUF):
                j = (g * NBUF + b) * ROWS
                gathers.append(
                    pltpu.async_copy(
                        table_hbm.at[idx_v.at[pl.ds(j, ROWS)]], buf.at[b], sem_g
                    )
                )
            writes = []
            for b in range(NBUF):
                j = (g * NBUF + b) * ROWS
                gathers[b].wait()
                writes.append(
                    pltpu.async_copy(
                        buf.at[b], out_hbm.at[pl.ds(base + j, ROWS)], sem_w
                    )
                )
            for b in range(NBUF):
                writes[b].wait()
            return carry

        lax.fori_loop(0, n_ch // NBUF, group, 0)

    out = emb(x_resh, table)
    return out.reshape(B, H, D)


# speedup vs baseline: 1.2233x; 1.2192x over previous
import functools

import jax
import jax.numpy as jnp
from jax import lax
from jax.experimental import pallas as pl
from jax.experimental.pallas import tpu as pltpu
from jax.experimental.pallas import tpu_sc as plsc

NW = 32
ROWS = 128
NBUF = 5


def kernel(x, table):
    B, H = x.shape
    V, D = table.shape
    DW = 128
    total = B * H
    per_w = total // NW
    n_ch = per_w // ROWS

    tw = jnp.pad(table, ((0, 0), (0, DW - D)))
    x_resh = x.reshape(NW, per_w).astype(jnp.int32)
    mesh = plsc.VectorSubcoreMesh(core_axis_name="c", subcore_axis_name="s")

    @functools.partial(
        pl.kernel,
        out_type=jax.ShapeDtypeStruct((total, DW), jnp.float32),
        mesh=mesh,
        scratch_types=[
            pltpu.VMEM((per_w,), jnp.int32),
            pltpu.VMEM((NBUF, ROWS, DW), jnp.float32),
            pltpu.SemaphoreType.DMA,
            pltpu.SemaphoreType.DMA,
        ],
        compiler_params=pltpu.CompilerParams(use_tc_tiling_on_sc=False),
    )
    def emb(x_hbm, table_hbm, out_hbm, idx_v, buf, sem_g, sem_w):
        wid = lax.axis_index("s") * 2 + lax.axis_index("c")
        base = wid * per_w
        pltpu.sync_copy(x_hbm.at[wid], idx_v)

        def group(g, carry):
            gathers = []
            for b in range(NBUF):
                j = (g * NBUF + b) * ROWS
                gathers.append(
                    pltpu.async_copy(
                        table_hbm.at[idx_v.at[pl.ds(j, ROWS)]], buf.at[b], sem_g
                    )
                )
            writes = []
            for b in range(NBUF):
                j = (g * NBUF + b) * ROWS
                gathers[b].wait()
                writes.append(
                    pltpu.async_copy(
                        buf.at[b], out_hbm.at[pl.ds(base + j, ROWS)], sem_w
                    )
                )
            for b in range(NBUF):
                writes[b].wait()
            return carry

        lax.fori_loop(0, n_ch // NBUF, group, 0)

    out = emb(x_resh, tw)
    return out[:, :D].reshape(B, H, D)
